# baseline (speedup 1.0000x reference)
import functools
import jax
import jax.numpy as jnp
from jax import lax
from jax.experimental import pallas as pl
from jax.experimental.pallas import tpu as pltpu
from jax.experimental.pallas import tpu_sc as plsc

_NU, _NI, _NB, _D = 50000, 40000, 20000, 64
_OFF_I = _NU
_OFF_U2 = _NU + _NI
_OFF_B = _OFF_U2 + _NU
_NTOT = _OFF_B + _NB
_NPAD = 163840
_EPS = 1e-8
_SENT = 1 << 28
_B = 2048
_G = 256
_ZR = 256
_NSUB = 16
_NSLAB = 4
_W = 16

_DSEG0 = (0, 50048)
_DSEG1 = (0, 50048, 70080)
_DEG_CHUNK = 40960
_DEG_ROWS = 3 * _DEG_CHUNK
_DW = 8


def _pad_edges(x, blk=_B * _NSUB):
    e = x.shape[0]
    ep = ((e + blk - 1) // blk) * blk
    return jnp.concatenate([x, jnp.full((ep - e,), _SENT, jnp.int32)])


def _scale_body(d_ref, o_ref):
    o_ref[...] = 1.0 / (jnp.sqrt(d_ref[...]) + _EPS)


def _inv_body(d_ref, o_ref):
    o_ref[...] = 1.0 / (d_ref[...] + _EPS)


def _ew_1d(body, x, rows=128):
    n = x.shape[0]
    assert (n // 128) % rows == 0, n
    x2 = x.reshape(n // 128, 128)
    out = pl.pallas_call(
        body,
        out_shape=jax.ShapeDtypeStruct((n // 128, 128), jnp.float32),
        grid=(n // 128 // rows,),
        in_specs=[pl.BlockSpec((rows, 128), lambda i: (i, 0))],
        out_specs=pl.BlockSpec((rows, 128), lambda i: (i, 0)),
    )(x2)
    return out.reshape(n)


def _mul_body(x_ref, s_ref, o_ref):
    o_ref[...] = x_ref[...] * s_ref[...]


def _rowscale(x, s, rows=512):
    n = x.shape[0]
    return pl.pallas_call(
        _mul_body,
        out_shape=jax.ShapeDtypeStruct((n, _D), jnp.float32),
        grid=(n // rows,),
        in_specs=[pl.BlockSpec((rows, _D), lambda i: (i, 0)),
                  pl.BlockSpec((rows, 1), lambda i: (i, 0))],
        out_specs=pl.BlockSpec((rows, _D), lambda i: (i, 0)),
    )(x, s.reshape(n, 1))


def _layer_body(inv_l, h_ref, s_ref, acc_ref, acc_o_ref, g_o_ref):
    s = s_ref[...]
    f = h_ref[...] * s * inv_l
    nrm = jnp.maximum(jnp.sqrt(jnp.sum(f * f, axis=1, keepdims=True)), 1e-12)
    acc_o_ref[...] = acc_ref[...] + f / nrm
    g_o_ref[...] = f * s


def _layer_update(h, s, acc, inv_l, rows=512):
    n = h.shape[0]
    return pl.pallas_call(
        functools.partial(_layer_body, inv_l),
        out_shape=(jax.ShapeDtypeStruct((n, _D), jnp.float32),
                   jax.ShapeDtypeStruct((n, _D), jnp.float32)),
        grid=(n // rows,),
        in_specs=[pl.BlockSpec((rows, _D), lambda i: (i, 0)),
                  pl.BlockSpec((rows, 1), lambda i: (i, 0)),
                  pl.BlockSpec((rows, _D), lambda i: (i, 0))],
        out_specs=(pl.BlockSpec((rows, _D), lambda i: (i, 0)),
                   pl.BlockSpec((rows, _D), lambda i: (i, 0))),
    )(h, s.reshape(n, 1), acc)


def _spmm_body(phases, acc_rows, table, *args):
    nin = len(args) - 18
    idx_refs = args[:nin]
    out = args[nin]
    (ebr, ebc, st_s, st_d, zb, acc, semg, sems, ebsem) = \
        args[nin + 1:nin + 10]
    rbufs = args[nin + 10:nin + 14]
    sixb = args[nin + 14:nin + 18]

    core = lax.axis_index("c")
    sub = lax.axis_index("s")
    zeros16 = jnp.zeros((16,), jnp.float32)

    def zi(k, _):
        zb[k, pl.ds(0, 16)] = zeros16
        return 0
    lax.fori_loop(0, _ZR, zi, 0)

    for p in range(2):
        slab = 2 * core + p
        for (ridx, cidx, bidir, nrowp, wb_base, d0, s0, d1, s1) in phases:
            rps = nrowp // _NSUB
            pad_slot = nrowp
            r_ref = idx_refs[ridx]
            c_ref = idx_refs[cidx]
            epb = 2 * _B if bidir else _B
            ngrp = epb // _G

            def za(q, _):
                pltpu.sync_copy(zb, acc.at[pl.ds(sub * rps + q * _ZR, _ZR)])
                return 0
            lax.fori_loop(0, rps // _ZR, za, 0)

            @pl.when(sub == 0)
            def _():
                pltpu.sync_copy(zb.at[pl.ds(0, 1)],
                                acc.at[pl.ds(pad_slot, 1)])
            plsc.subcore_barrier()

            eps_ = r_ref.shape[0] // _NSUB
            base = sub * eps_
            nblk = eps_ // _B

            pltpu.async_copy(r_ref.at[pl.ds(base, _B)], ebr.at[0], ebsem)
            pltpu.async_copy(c_ref.at[pl.ds(base, _B)], ebc.at[0], ebsem)

            def blk(b, _, r_ref=r_ref, c_ref=c_ref, base=base, bidir=bidir,
                    d0=d0, s0=s0, d1=d1, s1=s1, pad_slot=pad_slot,
                    ngrp=ngrp, slab=slab, nblk=nblk):
                par = lax.rem(b, 2)
                pltpu.make_async_copy(
                    r_ref.at[pl.ds(base + b * _B, _B)], ebr.at[par],
                    ebsem).wait()
                pltpu.make_async_copy(
                    c_ref.at[pl.ds(base + b * _B, _B)], ebc.at[par],
                    ebsem).wait()

                @pl.when(b + 1 < nblk)
                def _():
                    nb = base + (b + 1) * _B
                    pltpu.async_copy(r_ref.at[pl.ds(nb, _B)],
                                     ebr.at[1 - par], ebsem)
                    pltpu.async_copy(c_ref.at[pl.ds(nb, _B)],
                                     ebc.at[1 - par], ebsem)

                def vreg(k, _2):
                    r = ebr[par, pl.ds(k * 16, 16)]
                    c = ebc[par, pl.ds(k * 16, 16)]
                    ok = r < _SENT
                    if bidir:
                        st_d[pl.ds(32 * k, 16)] = jnp.where(
                            ok, r + d0, pad_slot)
                        st_s[pl.ds(32 * k, 16)] = jnp.where(ok, c + s0, 0)
                        st_d[pl.ds(32 * k + 16, 16)] = jnp.where(
                            ok, c + d1, pad_slot)
                        st_s[pl.ds(32 * k + 16, 16)] = jnp.where(
                            ok, r + s1, 0)
                    else:
                        st_d[pl.ds(16 * k, 16)] = jnp.where(
                            ok, r + d0, pad_slot)
                        st_s[pl.ds(16 * k, 16)] = jnp.where(ok, c + s0, 0)
                    return 0
                lax.fori_loop(0, _B // 16, vreg, 0)

                gets = [None] * ngrp
                puts = [None] * ngrp
                for q in range(ngrp + 2):
                    if q < ngrp:
                        sl = q % 4
                        if q >= 4:
                            puts[q - 4].wait()
                        gets[q] = pltpu.async_copy(
                            table.at[slab].at[st_s.at[pl.ds(q * _G, _G)]],
                            rbufs[sl], semg)
                        for t in range(_G // 16):
                            sixb[sl][pl.ds(t * 16, 16)] = st_d[
                                pl.ds(q * _G + t * 16, 16)]
                    if q >= 2:
                        j = q - 2
                        sl2 = j % 4
                        gets[j].wait()
                        puts[j] = pltpu.async_copy(
                            rbufs[sl2], acc.at[sixb[sl2]], sems, add=True)
                for j in range(ngrp - 4, ngrp):
                    puts[j].wait()
                return 0
            lax.fori_loop(0, eps_ // _B, blk, 0)

            plsc.subcore_barrier()
            pltpu.sync_copy(
                acc.at[pl.ds(sub * rps, rps)],
                out.at[slab].at[pl.ds(wb_base + sub * rps, rps)])
            plsc.subcore_barrier()


def _sc_spmm(table3, edge_arrays, phases, acc_rows, n_out):
    mesh = plsc.VectorSubcoreMesh(core_axis_name="c", subcore_axis_name="s")
    k = pl.kernel(
        functools.partial(_spmm_body, phases, acc_rows),
        out_type=jax.ShapeDtypeStruct((_NSLAB, n_out, _W), jnp.float32),
        mesh=mesh,
        compiler_params=pltpu.CompilerParams(use_tc_tiling_on_sc=False),
        scratch_types=[
            pltpu.VMEM((2, _B), jnp.int32),
            pltpu.VMEM((2, _B), jnp.int32),
            pltpu.VMEM((2 * _B,), jnp.int32),
            pltpu.VMEM((2 * _B,), jnp.int32),
            pltpu.VMEM((_ZR, _W), jnp.float32),
            pltpu.VMEM_SHARED((acc_rows, _W), jnp.float32),
            pltpu.SemaphoreType.DMA,
            pltpu.SemaphoreType.DMA,
            pltpu.SemaphoreType.DMA,
        ] + [pltpu.VMEM((_G, _W), jnp.float32)] * 4
          + [pltpu.VMEM((_G,), jnp.int32)] * 4,
    )
    return k(table3, *edge_arrays)


def _deg_body(ui_u, ui_i, ub_u, ub_b, bi_b, ones8, out, st_d, ebr, onesb,
              zb8, acc, sems, ebsem, *sixb):
    core = lax.axis_index("c")
    sub = lax.axis_index("s")
    rps = _DEG_CHUNK // _NSUB

    pltpu.sync_copy(ones8.at[0].at[pl.ds(0, _G)], onesb)
    pltpu.sync_copy(ones8.at[1].at[pl.ds(0, _ZR)], zb8)

    for chunk in range(3):
        lo = chunk * _DEG_CHUNK

        def za(q, _):
            pltpu.sync_copy(zb8, acc.at[pl.ds(sub * rps + q * _ZR, _ZR)])
            return 0
        lax.fori_loop(0, rps // _ZR, za, 0)

        @pl.when(sub == 0)
        def _():
            pltpu.sync_copy(zb8.at[pl.ds(0, 1)],
                            acc.at[pl.ds(_DEG_CHUNK, 1)])
        plsc.subcore_barrier()

        def count(idx_ref, seg_off):
            eps_ = idx_ref.shape[0] // _NSUB
            base = sub * eps_
            nblk = eps_ // _B
            ngrp = _B // _G

            pltpu.async_copy(idx_ref.at[pl.ds(base, _B)], ebr.at[0], ebsem)

            def blk(b, _, idx_ref=idx_ref, seg_off=seg_off, base=base,
                    nblk=nblk, ngrp=ngrp):
                par = lax.rem(b, 2)
                pltpu.make_async_copy(
                    idx_ref.at[pl.ds(base + b * _B, _B)], ebr.at[par],
                    ebsem).wait()

                @pl.when(b + 1 < nblk)
                def _():
                    pltpu.async_copy(
                        idx_ref.at[pl.ds(base + (b + 1) * _B, _B)],
                        ebr.at[1 - par], ebsem)

                def vreg(k, _2):
                    v = ebr[par, pl.ds(k * 16, 16)]
                    d = v + (seg_off - lo)
                    ok = (v < _SENT) & (d >= 0) & (d < _DEG_CHUNK)
                    st_d[pl.ds(16 * k, 16)] = jnp.where(ok, d, _DEG_CHUNK)
                    return 0
                lax.fori_loop(0, _B // 16, vreg, 0)

                puts = [None] * ngrp
                for q in range(ngrp):
                    sl = q % 4
                    if q >= 4:
                        puts[q - 4].wait()
                    for t in range(_G // 16):
                        sixb[sl][pl.ds(t * 16, 16)] = st_d[
                            pl.ds(q * _G + t * 16, 16)]
                    puts[q] = pltpu.async_copy(
                        onesb, acc.at[sixb[sl]], sems, add=True)
                for j in range(ngrp - 4, ngrp):
                    puts[j].wait()
                return 0
            lax.fori_loop(0, nblk, blk, 0)

        @pl.when(core == 0)
        def _():
            count(ui_u, _DSEG0[0])
            count(ui_i, _DSEG0[1])

        @pl.when(core == 1)
        def _():
            count(ub_u, _DSEG1[0])
            count(ub_b, _DSEG1[1])
            count(bi_b, _DSEG1[2])

        plsc.subcore_barrier()
        pltpu.sync_copy(acc.at[pl.ds(sub * rps, rps)],
                        out.at[core].at[pl.ds(lo + sub * rps, rps)])
        plsc.subcore_barrier()


def _sc_degrees(ui_u, ui_i, ub_u, ub_b, bi_b, ones8):
    mesh = plsc.VectorSubcoreMesh(core_axis_name="c", subcore_axis_name="s")
    k = pl.kernel(
        _deg_body,
        out_type=jax.ShapeDtypeStruct((2, _DEG_ROWS, _DW), jnp.float32),
        mesh=mesh,
        compiler_params=pltpu.CompilerParams(use_tc_tiling_on_sc=False),
        scratch_types=[
            pltpu.VMEM((_B,), jnp.int32),
            pltpu.VMEM((2, _B), jnp.int32),
            pltpu.VMEM((_G, _DW), jnp.float32),
            pltpu.VMEM((_ZR, _DW), jnp.float32),
            pltpu.VMEM_SHARED((_DEG_CHUNK + 1, _DW), jnp.float32),
            pltpu.SemaphoreType.DMA,
            pltpu.SemaphoreType.DMA,
        ] + [pltpu.VMEM((_G,), jnp.int32)] * 4,
    )
    return k(ui_u, ui_i, ub_u, ub_b, bi_b, ones8)


def _to_slabs(x):
    return x.reshape(x.shape[0], _NSLAB, _W).transpose(1, 0, 2)


def _from_slabs(x3):
    return x3.transpose(1, 0, 2).reshape(x3.shape[1], _D)


_PHASES_MAIN = [
    (0, 1, True, 90112, 0, 0, _OFF_I, _OFF_I, 0),
    (2, 3, True, 73728, _OFF_U2, 0, _OFF_B, _NU, _OFF_U2),
]

_PHASES_BI = [
    (0, 1, False, 20480, 0, 0, _OFF_I, None, None),
]


def kernel(users_feature, bundles_feature, items_feature, ui_u, ui_i, ub_u, ub_b, bi_b, bi_i):
    ui_u, ui_i = _pad_edges(ui_u), _pad_edges(ui_i)
    ub_u, ub_b = _pad_edges(ub_u), _pad_edges(ub_b)
    bi_b, bi_i = _pad_edges(bi_b), _pad_edges(bi_i)

    z8 = users_feature[:8192, :_DW] * 0.0
    ones8 = jnp.stack([z8 + 1.0, z8])
    degw = _sc_degrees(ui_u, ui_i, ub_u, ub_b, bi_b, ones8)
    d0 = degw[0, :, 0]
    d1 = degw[1, :, 0]
    s_all = _ew_1d(_scale_body, jnp.concatenate([
        d0[0:_NU], d0[_DSEG0[1]:_DSEG0[1] + _NI],
        d1[0:_NU], d1[_DSEG1[1]:_DSEG1[1] + _NB],
        jnp.zeros((_NPAD - _NTOT,), jnp.float32),
    ]))
    inv_bs = _ew_1d(_inv_body, jnp.concatenate([
        d1[_DSEG1[2]:_DSEG1[2] + _NB],
        jnp.zeros((20480 - _NB,), jnp.float32),
    ]), rows=160)

    table0 = jnp.concatenate([
        users_feature, items_feature, users_feature, bundles_feature,
        jnp.zeros((_NPAD - _NTOT, _D), jnp.float32),
    ])
    acc = table0
    g = _rowscale(table0, s_all)

    for i in range(2):
        h3 = _sc_spmm(_to_slabs(g), [ui_u, ui_i, ub_u, ub_b], _PHASES_MAIN,
                      acc_rows=90113, n_out=_NPAD)
        acc, g = _layer_update(_from_slabs(h3), s_all, acc, 1.0 / (i + 2))

    hb3 = _sc_spmm(_to_slabs(acc), [bi_b, bi_i], _PHASES_BI,
                   acc_rows=20481, n_out=20480)
    il_bundles = _rowscale(_from_slabs(hb3), inv_bs)[:_NB]

    return (acc[:_NU], acc[_OFF_U2:_OFF_U2 + _NU], il_bundles,
            acc[_OFF_B:_OFF_B + _NB])


# speedup vs baseline: 1.6586x; 1.6586x over previous
import functools
import jax
import jax.numpy as jnp
from jax import lax
from jax.experimental import pallas as pl
from jax.experimental.pallas import tpu as pltpu
from jax.experimental.pallas import tpu_sc as plsc

_NU, _NI, _NB, _D = 50000, 40000, 20000, 64
_OFF_I = _NU
_OFF_U2 = _NU + _NI
_OFF_B = _OFF_U2 + _NU
_NTOT = _OFF_B + _NB
_NPAD = 163840
_EPS = 1e-8
_SENT = 1 << 28
_B = 2048
_G = 256
_ZR = 256
_NSUB = 16
_NSLAB = 4
_W = 16

_DSEG0 = (0, 50048)
_DSEG1 = (0, 50048, 70080)
_DEG_CHUNK = 40960
_DEG_ROWS = 3 * _DEG_CHUNK
_DW = 8


def _pad_edges(x, blk=_B * _NSUB):
    e = x.shape[0]
    ep = ((e + blk - 1) // blk) * blk
    return jnp.concatenate([x, jnp.full((ep - e,), _SENT, jnp.int32)])


def _scale_body(d_ref, o_ref):
    o_ref[...] = 1.0 / (jnp.sqrt(d_ref[...]) + _EPS)


def _inv_body(d_ref, o_ref):
    o_ref[...] = 1.0 / (d_ref[...] + _EPS)


def _ew_1d(body, x, rows=128):
    n = x.shape[0]
    assert (n // 128) % rows == 0, n
    x2 = x.reshape(n // 128, 128)
    out = pl.pallas_call(
        body,
        out_shape=jax.ShapeDtypeStruct((n // 128, 128), jnp.float32),
        grid=(n // 128 // rows,),
        in_specs=[pl.BlockSpec((rows, 128), lambda i: (i, 0))],
        out_specs=pl.BlockSpec((rows, 128), lambda i: (i, 0)),
    )(x2)
    return out.reshape(n)


def _mul_body(x_ref, s_ref, o_ref):
    o_ref[...] = x_ref[...] * s_ref[...]


def _rowscale(x, s, rows=512):
    n = x.shape[0]
    return pl.pallas_call(
        _mul_body,
        out_shape=jax.ShapeDtypeStruct((n, _D), jnp.float32),
        grid=(n // rows,),
        in_specs=[pl.BlockSpec((rows, _D), lambda i: (i, 0)),
                  pl.BlockSpec((rows, 1), lambda i: (i, 0))],
        out_specs=pl.BlockSpec((rows, _D), lambda i: (i, 0)),
    )(x, s.reshape(n, 1))


def _layer_body(inv_l, h_ref, s_ref, acc_ref, acc_o_ref, g_o_ref):
    s = s_ref[...]
    f = h_ref[...] * s * inv_l
    nrm = jnp.maximum(jnp.sqrt(jnp.sum(f * f, axis=1, keepdims=True)), 1e-12)
    acc_o_ref[...] = acc_ref[...] + f / nrm
    g_o_ref[...] = f * s


def _layer_update(h, s, acc, inv_l, rows=512):
    n = h.shape[0]
    return pl.pallas_call(
        functools.partial(_layer_body, inv_l),
        out_shape=(jax.ShapeDtypeStruct((n, _D), jnp.float32),
                   jax.ShapeDtypeStruct((n, _D), jnp.float32)),
        grid=(n // rows,),
        in_specs=[pl.BlockSpec((rows, _D), lambda i: (i, 0)),
                  pl.BlockSpec((rows, 1), lambda i: (i, 0)),
                  pl.BlockSpec((rows, _D), lambda i: (i, 0))],
        out_specs=(pl.BlockSpec((rows, _D), lambda i: (i, 0)),
                   pl.BlockSpec((rows, _D), lambda i: (i, 0))),
    )(h, s.reshape(n, 1), acc)


def _spmm_body(phases, acc_rows, table, *args):
    nin = len(args) - 18
    idx_refs = args[:nin]
    out = args[nin]
    (ebr, ebc, st_s, st_d, zb, acc, semg, sems, ebsem) = \
        args[nin + 1:nin + 10]
    rbufs = args[nin + 10:nin + 14]
    sixb = args[nin + 14:nin + 18]

    core = lax.axis_index("c")
    sub = lax.axis_index("s")
    zeros16 = jnp.zeros((16,), jnp.float32)

    def zi(k, _):
        zb[k, pl.ds(0, 16)] = zeros16
        return 0
    lax.fori_loop(0, _ZR, zi, 0)

    for p in range(2):
        slab = 2 * core + p
        for (ridx, cidx, bidir, nrowp, wb_base, d0, s0, d1, s1) in phases:
            rps = nrowp // _NSUB
            pad_slot = nrowp
            r_ref = idx_refs[ridx]
            c_ref = idx_refs[cidx]
            epb = 2 * _B if bidir else _B
            ngrp = epb // _G

            def za(q, _):
                pltpu.sync_copy(zb, acc.at[pl.ds(sub * rps + q * _ZR, _ZR)])
                return 0
            lax.fori_loop(0, rps // _ZR, za, 0)

            @pl.when(sub == 0)
            def _():
                pltpu.sync_copy(zb.at[pl.ds(0, 1)],
                                acc.at[pl.ds(pad_slot, 1)])
            plsc.subcore_barrier()

            eps_ = r_ref.shape[0] // _NSUB
            base = sub * eps_
            nblk = eps_ // _B

            pltpu.async_copy(r_ref.at[pl.ds(base, _B)], ebr.at[0], ebsem)
            pltpu.async_copy(c_ref.at[pl.ds(base, _B)], ebc.at[0], ebsem)

            def blk(b, _, r_ref=r_ref, c_ref=c_ref, base=base, bidir=bidir,
                    d0=d0, s0=s0, d1=d1, s1=s1, pad_slot=pad_slot,
                    ngrp=ngrp, slab=slab, nblk=nblk):
                par = lax.rem(b, 2)
                pltpu.make_async_copy(
                    r_ref.at[pl.ds(base + b * _B, _B)], ebr.at[par],
                    ebsem).wait()
                pltpu.make_async_copy(
                    c_ref.at[pl.ds(base + b * _B, _B)], ebc.at[par],
                    ebsem).wait()

                @pl.when(b + 1 < nblk)
                def _():
                    nb = base + (b + 1) * _B
                    pltpu.async_copy(r_ref.at[pl.ds(nb, _B)],
                                     ebr.at[1 - par], ebsem)
                    pltpu.async_copy(c_ref.at[pl.ds(nb, _B)],
                                     ebc.at[1 - par], ebsem)

                def vreg(k, _2):
                    r = ebr[par, pl.ds(k * 16, 16)]
                    c = ebc[par, pl.ds(k * 16, 16)]
                    ok = r < _SENT
                    pad = pad_slot + ((16 * k + lax.iota(jnp.int32, 16))
                                      & 2047)
                    if bidir:
                        st_d[pl.ds(32 * k, 16)] = jnp.where(ok, r + d0, pad)
                        st_s[pl.ds(32 * k, 16)] = jnp.where(ok, c + s0, 0)
                        st_d[pl.ds(32 * k + 16, 16)] = jnp.where(
                            ok, c + d1, pad)
                        st_s[pl.ds(32 * k + 16, 16)] = jnp.where(
                            ok, r + s1, 0)
                    else:
                        st_d[pl.ds(16 * k, 16)] = jnp.where(ok, r + d0, pad)
                        st_s[pl.ds(16 * k, 16)] = jnp.where(ok, c + s0, 0)
                    return 0
                lax.fori_loop(0, _B // 16, vreg, 0)

                gets = [None] * ngrp
                puts = [None] * ngrp
                for q in range(ngrp + 2):
                    if q < ngrp:
                        sl = q % 4
                        if q >= 4:
                            puts[q - 4].wait()
                        gets[q] = pltpu.async_copy(
                            table.at[slab].at[st_s.at[pl.ds(q * _G, _G)]],
                            rbufs[sl], semg)
                        for t in range(_G // 16):
                            sixb[sl][pl.ds(t * 16, 16)] = st_d[
                                pl.ds(q * _G + t * 16, 16)]
                    if q >= 2:
                        j = q - 2
                        sl2 = j % 4
                        gets[j].wait()
                        puts[j] = pltpu.async_copy(
                            rbufs[sl2], acc.at[sixb[sl2]], sems, add=True)
                for j in range(ngrp - 4, ngrp):
                    puts[j].wait()
                return 0
            lax.fori_loop(0, eps_ // _B, blk, 0)

            plsc.subcore_barrier()
            pltpu.sync_copy(
                acc.at[pl.ds(sub * rps, rps)],
                out.at[slab].at[pl.ds(wb_base + sub * rps, rps)])
            plsc.subcore_barrier()


def _sc_spmm(table3, edge_arrays, phases, acc_rows, n_out):
    mesh = plsc.VectorSubcoreMesh(core_axis_name="c", subcore_axis_name="s")
    k = pl.kernel(
        functools.partial(_spmm_body, phases, acc_rows),
        out_type=jax.ShapeDtypeStruct((_NSLAB, n_out, _W), jnp.float32),
        mesh=mesh,
        compiler_params=pltpu.CompilerParams(use_tc_tiling_on_sc=False),
        scratch_types=[
            pltpu.VMEM((2, _B), jnp.int32),
            pltpu.VMEM((2, _B), jnp.int32),
            pltpu.VMEM((2 * _B,), jnp.int32),
            pltpu.VMEM((2 * _B,), jnp.int32),
            pltpu.VMEM((_ZR, _W), jnp.float32),
            pltpu.VMEM_SHARED((acc_rows, _W), jnp.float32),
            pltpu.SemaphoreType.DMA,
            pltpu.SemaphoreType.DMA,
            pltpu.SemaphoreType.DMA,
        ] + [pltpu.VMEM((_G, _W), jnp.float32)] * 4
          + [pltpu.VMEM((_G,), jnp.int32)] * 4,
    )
    return k(table3, *edge_arrays)


def _deg_body(ui_u, ui_i, ub_u, ub_b, bi_b, ones8, out, st_d, ebr, onesb,
              zb8, acc, sems, ebsem, *sixb):
    core = lax.axis_index("c")
    sub = lax.axis_index("s")
    rps = _DEG_CHUNK // _NSUB

    pltpu.sync_copy(ones8.at[0].at[pl.ds(0, _G)], onesb)
    pltpu.sync_copy(ones8.at[1].at[pl.ds(0, _ZR)], zb8)

    for chunk in range(3):
        lo = chunk * _DEG_CHUNK

        def za(q, _):
            pltpu.sync_copy(zb8, acc.at[pl.ds(sub * rps + q * _ZR, _ZR)])
            return 0
        lax.fori_loop(0, rps // _ZR, za, 0)

        @pl.when(sub == 0)
        def _():
            pltpu.sync_copy(zb8.at[pl.ds(0, 1)],
                            acc.at[pl.ds(_DEG_CHUNK, 1)])
        plsc.subcore_barrier()

        def count(idx_ref, seg_off):
            eps_ = idx_ref.shape[0] // _NSUB
            base = sub * eps_
            nblk = eps_ // _B
            ngrp = _B // _G

            pltpu.async_copy(idx_ref.at[pl.ds(base, _B)], ebr.at[0], ebsem)

            def blk(b, _, idx_ref=idx_ref, seg_off=seg_off, base=base,
                    nblk=nblk, ngrp=ngrp):
                par = lax.rem(b, 2)
                pltpu.make_async_copy(
                    idx_ref.at[pl.ds(base + b * _B, _B)], ebr.at[par],
                    ebsem).wait()

                @pl.when(b + 1 < nblk)
                def _():
                    pltpu.async_copy(
                        idx_ref.at[pl.ds(base + (b + 1) * _B, _B)],
                        ebr.at[1 - par], ebsem)

                def vreg(k, _2):
                    v = ebr[par, pl.ds(k * 16, 16)]
                    d = v + (seg_off - lo)
                    ok = (v < _SENT) & (d >= 0) & (d < _DEG_CHUNK)
                    st_d[pl.ds(16 * k, 16)] = jnp.where(
                        ok, d, _DEG_CHUNK +
                        ((v + 16 * k + lax.iota(jnp.int32, 16)) & 2047))
                    return 0
                lax.fori_loop(0, _B // 16, vreg, 0)

                puts = [None] * ngrp
                for q in range(ngrp):
                    sl = q % 4
                    if q >= 4:
                        puts[q - 4].wait()
                    for t in range(_G // 16):
                        sixb[sl][pl.ds(t * 16, 16)] = st_d[
                            pl.ds(q * _G + t * 16, 16)]
                    puts[q] = pltpu.async_copy(
                        onesb, acc.at[sixb[sl]], sems, add=True)
                for j in range(ngrp - 4, ngrp):
                    puts[j].wait()
                return 0
            lax.fori_loop(0, nblk, blk, 0)

        @pl.when(core == 0)
        def _():
            count(ui_u, _DSEG0[0])
            count(ui_i, _DSEG0[1])

        @pl.when(core == 1)
        def _():
            count(ub_u, _DSEG1[0])
            count(ub_b, _DSEG1[1])
            count(bi_b, _DSEG1[2])

        plsc.subcore_barrier()
        pltpu.sync_copy(acc.at[pl.ds(sub * rps, rps)],
                        out.at[core].at[pl.ds(lo + sub * rps, rps)])
        plsc.subcore_barrier()


def _sc_degrees(ui_u, ui_i, ub_u, ub_b, bi_b, ones8):
    mesh = plsc.VectorSubcoreMesh(core_axis_name="c", subcore_axis_name="s")
    k = pl.kernel(
        _deg_body,
        out_type=jax.ShapeDtypeStruct((2, _DEG_ROWS, _DW), jnp.float32),
        mesh=mesh,
        compiler_params=pltpu.CompilerParams(use_tc_tiling_on_sc=False),
        scratch_types=[
            pltpu.VMEM((_B,), jnp.int32),
            pltpu.VMEM((2, _B), jnp.int32),
            pltpu.VMEM((_G, _DW), jnp.float32),
            pltpu.VMEM((_ZR, _DW), jnp.float32),
            pltpu.VMEM_SHARED((_DEG_CHUNK + 2048, _DW), jnp.float32),
            pltpu.SemaphoreType.DMA,
            pltpu.SemaphoreType.DMA,
        ] + [pltpu.VMEM((_G,), jnp.int32)] * 4,
    )
    return k(ui_u, ui_i, ub_u, ub_b, bi_b, ones8)


def _to_slabs(x):
    return x.reshape(x.shape[0], _NSLAB, _W).transpose(1, 0, 2)


def _from_slabs(x3):
    return x3.transpose(1, 0, 2).reshape(x3.shape[1], _D)


_PHASES_MAIN = [
    (0, 1, True, 90112, 0, 0, _OFF_I, _OFF_I, 0),
    (2, 3, True, 73728, _OFF_U2, 0, _OFF_B, _NU, _OFF_U2),
]

_PHASES_BI = [
    (0, 1, False, 20480, 0, 0, _OFF_I, None, None),
]


def kernel(users_feature, bundles_feature, items_feature, ui_u, ui_i, ub_u, ub_b, bi_b, bi_i):
    ui_u, ui_i = _pad_edges(ui_u), _pad_edges(ui_i)
    ub_u, ub_b = _pad_edges(ub_u), _pad_edges(ub_b)
    bi_b, bi_i = _pad_edges(bi_b), _pad_edges(bi_i)

    z8 = users_feature[:8192, :_DW] * 0.0
    ones8 = jnp.stack([z8 + 1.0, z8])
    degw = _sc_degrees(ui_u, ui_i, ub_u, ub_b, bi_b, ones8)
    d0 = degw[0, :, 0]
    d1 = degw[1, :, 0]
    s_all = _ew_1d(_scale_body, jnp.concatenate([
        d0[0:_NU], d0[_DSEG0[1]:_DSEG0[1] + _NI],
        d1[0:_NU], d1[_DSEG1[1]:_DSEG1[1] + _NB],
        jnp.zeros((_NPAD - _NTOT,), jnp.float32),
    ]))
    inv_bs = _ew_1d(_inv_body, jnp.concatenate([
        d1[_DSEG1[2]:_DSEG1[2] + _NB],
        jnp.zeros((20480 - _NB,), jnp.float32),
    ]), rows=160)

    table0 = jnp.concatenate([
        users_feature, items_feature, users_feature, bundles_feature,
        jnp.zeros((_NPAD - _NTOT, _D), jnp.float32),
    ])
    acc = table0
    g = _rowscale(table0, s_all)

    for i in range(2):
        h3 = _sc_spmm(_to_slabs(g), [ui_u, ui_i, ub_u, ub_b], _PHASES_MAIN,
                      acc_rows=92160, n_out=_NPAD)
        acc, g = _layer_update(_from_slabs(h3), s_all, acc, 1.0 / (i + 2))

    hb3 = _sc_spmm(_to_slabs(acc), [bi_b, bi_i], _PHASES_BI,
                   acc_rows=22528, n_out=20480)
    il_bundles = _rowscale(_from_slabs(hb3), inv_bs)[:_NB]

    return (acc[:_NU], acc[_OFF_U2:_OFF_U2 + _NU], il_bundles,
            acc[_OFF_B:_OFF_B + _NB])


# speedup vs baseline: 1.6612x; 1.0016x over previous
import functools
import jax
import jax.numpy as jnp
from jax import lax
from jax.experimental import pallas as pl
from jax.experimental.pallas import tpu as pltpu
from jax.experimental.pallas import tpu_sc as plsc

_NU, _NI, _NB, _D = 50000, 40000, 20000, 64
_OFF_I = _NU
_OFF_U2 = _NU + _NI
_OFF_B = _OFF_U2 + _NU
_NTOT = _OFF_B + _NB
_NPAD = 163840
_EPS = 1e-8
_SENT = 1 << 28
_B = 2048
_G = 256
_ZR = 256
_GD = 256
_NSUB = 16
_NSLAB = 4
_W = 16

_DSEG0 = (0, 50048)
_DSEG1 = (0, 50048, 70080)
_DEG_CHUNK = 40960
_DEG_ROWS = 3 * _DEG_CHUNK
_DW = 8


def _pad_edges(x, blk=_B * _NSUB):
    e = x.shape[0]
    ep = ((e + blk - 1) // blk) * blk
    return jnp.concatenate([x, jnp.full((ep - e,), _SENT, jnp.int32)])


def _scale_body(d_ref, o_ref):
    o_ref[...] = 1.0 / (jnp.sqrt(d_ref[...]) + _EPS)


def _inv_body(d_ref, o_ref):
    o_ref[...] = 1.0 / (d_ref[...] + _EPS)


def _ew_1d(body, x, rows=128):
    n = x.shape[0]
    assert (n // 128) % rows == 0, n
    x2 = x.reshape(n // 128, 128)
    out = pl.pallas_call(
        body,
        out_shape=jax.ShapeDtypeStruct((n // 128, 128), jnp.float32),
        grid=(n // 128 // rows,),
        in_specs=[pl.BlockSpec((rows, 128), lambda i: (i, 0))],
        out_specs=pl.BlockSpec((rows, 128), lambda i: (i, 0)),
    )(x2)
    return out.reshape(n)


def _mul_body(x_ref, s_ref, o_ref):
    o_ref[...] = x_ref[...] * s_ref[...]


def _rowscale(x, s, rows=512):
    n = x.shape[0]
    return pl.pallas_call(
        _mul_body,
        out_shape=jax.ShapeDtypeStruct((n, _D), jnp.float32),
        grid=(n // rows,),
        in_specs=[pl.BlockSpec((rows, _D), lambda i: (i, 0)),
                  pl.BlockSpec((rows, 1), lambda i: (i, 0))],
        out_specs=pl.BlockSpec((rows, _D), lambda i: (i, 0)),
    )(x, s.reshape(n, 1))


def _layer_body(inv_l, h_ref, s_ref, acc_ref, acc_o_ref, g_o_ref):
    s = s_ref[...]
    f = h_ref[...] * s * inv_l
    nrm = jnp.maximum(jnp.sqrt(jnp.sum(f * f, axis=1, keepdims=True)), 1e-12)
    acc_o_ref[...] = acc_ref[...] + f / nrm
    g_o_ref[...] = f * s


def _layer_update(h, s, acc, inv_l, rows=512):
    n = h.shape[0]
    return pl.pallas_call(
        functools.partial(_layer_body, inv_l),
        out_shape=(jax.ShapeDtypeStruct((n, _D), jnp.float32),
                   jax.ShapeDtypeStruct((n, _D), jnp.float32)),
        grid=(n // rows,),
        in_specs=[pl.BlockSpec((rows, _D), lambda i: (i, 0)),
                  pl.BlockSpec((rows, 1), lambda i: (i, 0)),
                  pl.BlockSpec((rows, _D), lambda i: (i, 0))],
        out_specs=(pl.BlockSpec((rows, _D), lambda i: (i, 0)),
                   pl.BlockSpec((rows, _D), lambda i: (i, 0))),
    )(h, s.reshape(n, 1), acc)


def _spmm_body(phases, acc_rows, table, *args):
    nin = len(args) - 18
    idx_refs = args[:nin]
    out = args[nin]
    (ebr, ebc, st_s, st_d, zb, acc, semg, sems, ebsem) = \
        args[nin + 1:nin + 10]
    rbufs = args[nin + 10:nin + 14]
    sixb = args[nin + 14:nin + 18]

    core = lax.axis_index("c")
    sub = lax.axis_index("s")
    zeros16 = jnp.zeros((16,), jnp.float32)

    def zi(k, _):
        zb[k, pl.ds(0, 16)] = zeros16
        return 0
    lax.fori_loop(0, _ZR, zi, 0)

    for p in range(2):
        slab = 2 * core + p
        for (ridx, cidx, bidir, nrowp, wb_base, d0, s0, d1, s1) in phases:
            rps = nrowp // _NSUB
            pad_slot = nrowp
            r_ref = idx_refs[ridx]
            c_ref = idx_refs[cidx]
            epb = 2 * _B if bidir else _B
            ngrp = epb // _G

            def za(q, _):
                pltpu.sync_copy(zb, acc.at[pl.ds(sub * rps + q * _ZR, _ZR)])
                return 0
            lax.fori_loop(0, rps // _ZR, za, 0)

            @pl.when(sub == 0)
            def _():
                pltpu.sync_copy(zb.at[pl.ds(0, 1)],
                                acc.at[pl.ds(pad_slot, 1)])
            plsc.subcore_barrier()

            eps_ = r_ref.shape[0] // _NSUB
            base = sub * eps_
            nblk = eps_ // _B

            pltpu.async_copy(r_ref.at[pl.ds(base, _B)], ebr.at[0], ebsem)
            pltpu.async_copy(c_ref.at[pl.ds(base, _B)], ebc.at[0], ebsem)

            def blk(b, _, r_ref=r_ref, c_ref=c_ref, base=base, bidir=bidir,
                    d0=d0, s0=s0, d1=d1, s1=s1, pad_slot=pad_slot,
                    ngrp=ngrp, slab=slab, nblk=nblk):
                par = lax.rem(b, 2)
                pltpu.make_async_copy(
                    r_ref.at[pl.ds(base + b * _B, _B)], ebr.at[par],
                    ebsem).wait()
                pltpu.make_async_copy(
                    c_ref.at[pl.ds(base + b * _B, _B)], ebc.at[par],
                    ebsem).wait()

                @pl.when(b + 1 < nblk)
                def _():
                    nb = base + (b + 1) * _B
                    pltpu.async_copy(r_ref.at[pl.ds(nb, _B)],
                                     ebr.at[1 - par], ebsem)
                    pltpu.async_copy(c_ref.at[pl.ds(nb, _B)],
                                     ebc.at[1 - par], ebsem)

                def vreg(k, _2):
                    r = ebr[par, pl.ds(k * 16, 16)]
                    c = ebc[par, pl.ds(k * 16, 16)]
                    ok = r < _SENT
                    pad = pad_slot + ((16 * k + lax.iota(jnp.int32, 16))
                                      & 2047)
                    if bidir:
                        st_d[pl.ds(32 * k, 16)] = jnp.where(ok, r + d0, pad)
                        st_s[pl.ds(32 * k, 16)] = jnp.where(ok, c + s0, 0)
                        st_d[pl.ds(32 * k + 16, 16)] = jnp.where(
                            ok, c + d1, pad)
                        st_s[pl.ds(32 * k + 16, 16)] = jnp.where(
                            ok, r + s1, 0)
                    else:
                        st_d[pl.ds(16 * k, 16)] = jnp.where(ok, r + d0, pad)
                        st_s[pl.ds(16 * k, 16)] = jnp.where(ok, c + s0, 0)
                    return 0
                lax.fori_loop(0, _B // 16, vreg, 0)

                gets = [None] * ngrp
                puts = [None] * ngrp
                for q in range(ngrp + 2):
                    if q < ngrp:
                        sl = q % 4
                        if q >= 4:
                            puts[q - 4].wait()
                        gets[q] = pltpu.async_copy(
                            table.at[slab].at[st_s.at[pl.ds(q * _G, _G)]],
                            rbufs[sl], semg)
                        for t in range(_G // 16):
                            sixb[sl][pl.ds(t * 16, 16)] = st_d[
                                pl.ds(q * _G + t * 16, 16)]
                    if q >= 2:
                        j = q - 2
                        sl2 = j % 4
                        gets[j].wait()
                        puts[j] = pltpu.async_copy(
                            rbufs[sl2], acc.at[sixb[sl2]], sems, add=True)
                for j in range(ngrp - 4, ngrp):
                    puts[j].wait()
                return 0
            lax.fori_loop(0, eps_ // _B, blk, 0)

            plsc.subcore_barrier()
            pltpu.sync_copy(
                acc.at[pl.ds(sub * rps, rps)],
                out.at[slab].at[pl.ds(wb_base + sub * rps, rps)])
            plsc.subcore_barrier()


def _sc_spmm(table3, edge_arrays, phases, acc_rows, n_out):
    mesh = plsc.VectorSubcoreMesh(core_axis_name="c", subcore_axis_name="s")
    k = pl.kernel(
        functools.partial(_spmm_body, phases, acc_rows),
        out_type=jax.ShapeDtypeStruct((_NSLAB, n_out, _W), jnp.float32),
        mesh=mesh,
        compiler_params=pltpu.CompilerParams(use_tc_tiling_on_sc=False),
        scratch_types=[
            pltpu.VMEM((2, _B), jnp.int32),
            pltpu.VMEM((2, _B), jnp.int32),
            pltpu.VMEM((2 * _B,), jnp.int32),
            pltpu.VMEM((2 * _B,), jnp.int32),
            pltpu.VMEM((_ZR, _W), jnp.float32),
            pltpu.VMEM_SHARED((acc_rows, _W), jnp.float32),
            pltpu.SemaphoreType.DMA,
            pltpu.SemaphoreType.DMA,
            pltpu.SemaphoreType.DMA,
        ] + [pltpu.VMEM((_G, _W), jnp.float32)] * 4
          + [pltpu.VMEM((_G,), jnp.int32)] * 4,
    )
    return k(table3, *edge_arrays)


def _deg_body(ui_u, ui_i, ub_u, ub_b, bi_b, ones8, out, st_d, ebr, onesb,
              zb8, acc, sems, ebsem, *sixb):
    core = lax.axis_index("c")
    sub = lax.axis_index("s")
    rps = _DEG_CHUNK // _NSUB

    pltpu.sync_copy(ones8.at[0].at[pl.ds(0, _GD)], onesb)
    pltpu.sync_copy(ones8.at[1].at[pl.ds(0, _ZR)], zb8)

    for chunk in range(3):
        lo = chunk * _DEG_CHUNK

        def za(q, _):
            pltpu.sync_copy(zb8, acc.at[pl.ds(sub * rps + q * _ZR, _ZR)])
            return 0
        lax.fori_loop(0, rps // _ZR, za, 0)

        @pl.when(sub == 0)
        def _():
            pltpu.sync_copy(zb8.at[pl.ds(0, 1)],
                            acc.at[pl.ds(_DEG_CHUNK, 1)])
        plsc.subcore_barrier()

        def count(idx_ref, seg_off):
            eps_ = idx_ref.shape[0] // _NSUB
            base = sub * eps_
            nblk = eps_ // _B
            ngrp = _B // _GD

            pltpu.async_copy(idx_ref.at[pl.ds(base, _B)], ebr.at[0], ebsem)

            def blk(b, _, idx_ref=idx_ref, seg_off=seg_off, base=base,
                    nblk=nblk, ngrp=ngrp):
                par = lax.rem(b, 2)
                pltpu.make_async_copy(
                    idx_ref.at[pl.ds(base + b * _B, _B)], ebr.at[par],
                    ebsem).wait()

                @pl.when(b + 1 < nblk)
                def _():
                    pltpu.async_copy(
                        idx_ref.at[pl.ds(base + (b + 1) * _B, _B)],
                        ebr.at[1 - par], ebsem)

                def vreg(k, _2):
                    v = ebr[par, pl.ds(k * 16, 16)]
                    d = v + (seg_off - lo)
                    ok = (v < _SENT) & (d >= 0) & (d < _DEG_CHUNK)
                    st_d[pl.ds(16 * k, 16)] = jnp.where(
                        ok, d, _DEG_CHUNK +
                        ((v + 16 * k + lax.iota(jnp.int32, 16)) & 2047))
                    return 0
                lax.fori_loop(0, _B // 16, vreg, 0)

                puts = [None] * ngrp
                for q in range(ngrp):
                    sl = q % 4
                    if q >= 4:
                        puts[q - 4].wait()
                    for t in range(_GD // 16):
                        sixb[sl][pl.ds(t * 16, 16)] = st_d[
                            pl.ds(q * _GD + t * 16, 16)]
                    puts[q] = pltpu.async_copy(
                        onesb, acc.at[sixb[sl]], sems, add=True)
                for j in range(ngrp - 4, ngrp):
                    puts[j].wait()
                return 0
            lax.fori_loop(0, nblk, blk, 0)

        @pl.when(core == 0)
        def _():
            count(ui_u, _DSEG0[0])
            count(ui_i, _DSEG0[1])

        @pl.when(core == 1)
        def _():
            count(ub_u, _DSEG1[0])
            count(ub_b, _DSEG1[1])
            count(bi_b, _DSEG1[2])

        plsc.subcore_barrier()
        pltpu.sync_copy(acc.at[pl.ds(sub * rps, rps)],
                        out.at[core].at[pl.ds(lo + sub * rps, rps)])
        plsc.subcore_barrier()


def _sc_degrees(ui_u, ui_i, ub_u, ub_b, bi_b, ones8):
    mesh = plsc.VectorSubcoreMesh(core_axis_name="c", subcore_axis_name="s")
    k = pl.kernel(
        _deg_body,
        out_type=jax.ShapeDtypeStruct((2, _DEG_ROWS, _DW), jnp.float32),
        mesh=mesh,
        compiler_params=pltpu.CompilerParams(use_tc_tiling_on_sc=False),
        scratch_types=[
            pltpu.VMEM((_B,), jnp.int32),
            pltpu.VMEM((2, _B), jnp.int32),
            pltpu.VMEM((_GD, _DW), jnp.float32),
            pltpu.VMEM((_ZR, _DW), jnp.float32),
            pltpu.VMEM_SHARED((_DEG_CHUNK + 2048, _DW), jnp.float32),
            pltpu.SemaphoreType.DMA,
            pltpu.SemaphoreType.DMA,
        ] + [pltpu.VMEM((_GD,), jnp.int32)] * 4,
    )
    return k(ui_u, ui_i, ub_u, ub_b, bi_b, ones8)


def _to_slabs(x):
    return x.reshape(x.shape[0], _NSLAB, _W).transpose(1, 0, 2)


def _from_slabs(x3):
    return x3.transpose(1, 0, 2).reshape(x3.shape[1], _D)


_PHASES_MAIN = [
    (0, 1, True, 90112, 0, 0, _OFF_I, _OFF_I, 0),
    (2, 3, True, 73728, _OFF_U2, 0, _OFF_B, _NU, _OFF_U2),
]

_PHASES_BI = [
    (0, 1, False, 20480, 0, 0, _OFF_I, None, None),
]


def kernel(users_feature, bundles_feature, items_feature, ui_u, ui_i, ub_u, ub_b, bi_b, bi_i):
    ui_u, ui_i = _pad_edges(ui_u), _pad_edges(ui_i)
    ub_u, ub_b = _pad_edges(ub_u), _pad_edges(ub_b)
    bi_b, bi_i = _pad_edges(bi_b), _pad_edges(bi_i)

    z8 = users_feature[:8192, :_DW] * 0.0
    ones8 = jnp.stack([z8 + 1.0, z8])
    degw = _sc_degrees(ui_u, ui_i, ub_u, ub_b, bi_b, ones8)
    d0 = degw[0, :, 0]
    d1 = degw[1, :, 0]
    s_all = _ew_1d(_scale_body, jnp.concatenate([
        d0[0:_NU], d0[_DSEG0[1]:_DSEG0[1] + _NI],
        d1[0:_NU], d1[_DSEG1[1]:_DSEG1[1] + _NB],
        jnp.zeros((_NPAD - _NTOT,), jnp.float32),
    ]))
    inv_bs = _ew_1d(_inv_body, jnp.concatenate([
        d1[_DSEG1[2]:_DSEG1[2] + _NB],
        jnp.zeros((20480 - _NB,), jnp.float32),
    ]), rows=160)

    table0 = jnp.concatenate([
        users_feature, items_feature, users_feature, bundles_feature,
        jnp.zeros((_NPAD - _NTOT, _D), jnp.float32),
    ])
    acc = table0
    g = _rowscale(table0, s_all)

    for i in range(2):
        h3 = _sc_spmm(_to_slabs(g), [ui_u, ui_i, ub_u, ub_b], _PHASES_MAIN,
                      acc_rows=92160, n_out=_NPAD)
        acc, g = _layer_update(_from_slabs(h3), s_all, acc, 1.0 / (i + 2))

    hb3 = _sc_spmm(_to_slabs(acc), [bi_b, bi_i], _PHASES_BI,
                   acc_rows=22528, n_out=20480)
    il_bundles = _rowscale(_from_slabs(hb3), inv_bs)[:_NB]

    return (acc[:_NU], acc[_OFF_U2:_OFF_U2 + _NU], il_bundles,
            acc[_OFF_B:_OFF_B + _NB])


# speedup vs baseline: 1.7318x; 1.0425x over previous
import functools
import jax
import jax.numpy as jnp
from jax import lax
from jax.experimental import pallas as pl
from jax.experimental.pallas import tpu as pltpu
from jax.experimental.pallas import tpu_sc as plsc

_NU, _NI, _NB, _D = 50000, 40000, 20000, 64
_OFF_I = _NU
_OFF_U2 = _NU + _NI
_OFF_B = _OFF_U2 + _NU
_NTOT = _OFF_B + _NB
_NPAD = 163840
_EPS = 1e-8
_SENT = 1 << 28
_B = 2048
_G = 256
_ZR = 256
_GD = 256
_NSUB = 16
_NSLAB = 4
_W = 16

_DSEG0 = (0, 50048)
_DSEG1 = (0, 50048, 70080)
_DEG_CHUNK = 40960
_DEG_ROWS = 3 * _DEG_CHUNK
_DW = 8


def _pad_edges(x, blk=_B * _NSUB):
    e = x.shape[0]
    ep = ((e + blk - 1) // blk) * blk
    return jnp.concatenate([x, jnp.full((ep - e,), _SENT, jnp.int32)])


def _scale_body(d_ref, o_ref):
    o_ref[...] = 1.0 / (jnp.sqrt(d_ref[...]) + _EPS)


def _inv_body(d_ref, o_ref):
    o_ref[...] = 1.0 / (d_ref[...] + _EPS)


def _ew_1d(body, x, rows=128):
    n = x.shape[0]
    assert (n // 128) % rows == 0, n
    x2 = x.reshape(n // 128, 128)
    out = pl.pallas_call(
        body,
        out_shape=jax.ShapeDtypeStruct((n // 128, 128), jnp.float32),
        grid=(n // 128 // rows,),
        in_specs=[pl.BlockSpec((rows, 128), lambda i: (i, 0))],
        out_specs=pl.BlockSpec((rows, 128), lambda i: (i, 0)),
    )(x2)
    return out.reshape(n)


def _mul_body(x_ref, s_ref, o_ref):
    o_ref[...] = x_ref[...] * s_ref[...]


def _rowscale(x, s, rows=512):
    n = x.shape[0]
    return pl.pallas_call(
        _mul_body,
        out_shape=jax.ShapeDtypeStruct((n, _D), jnp.float32),
        grid=(n // rows,),
        in_specs=[pl.BlockSpec((rows, _D), lambda i: (i, 0)),
                  pl.BlockSpec((rows, 1), lambda i: (i, 0))],
        out_specs=pl.BlockSpec((rows, _D), lambda i: (i, 0)),
    )(x, s.reshape(n, 1))


def _layer_body(inv_l, h_ref, s_ref, acc_ref, acc_o_ref, g_o_ref):
    s = s_ref[...]
    f = h_ref[...] * s * inv_l
    nrm = jnp.maximum(jnp.sqrt(jnp.sum(f * f, axis=1, keepdims=True)), 1e-12)
    acc_o_ref[...] = acc_ref[...] + f / nrm
    g_o_ref[...] = f * s


def _layer_update(h, s, acc, inv_l, rows=512):
    n = h.shape[0]
    return pl.pallas_call(
        functools.partial(_layer_body, inv_l),
        out_shape=(jax.ShapeDtypeStruct((n, _D), jnp.float32),
                   jax.ShapeDtypeStruct((n, _D), jnp.float32)),
        grid=(n // rows,),
        in_specs=[pl.BlockSpec((rows, _D), lambda i: (i, 0)),
                  pl.BlockSpec((rows, 1), lambda i: (i, 0)),
                  pl.BlockSpec((rows, _D), lambda i: (i, 0))],
        out_specs=(pl.BlockSpec((rows, _D), lambda i: (i, 0)),
                   pl.BlockSpec((rows, _D), lambda i: (i, 0))),
    )(h, s.reshape(n, 1), acc)


def _spmm_body(phases, acc_rows, table, *args):
    nin = len(args) - 18
    idx_refs = args[:nin]
    out = args[nin]
    (ebr, ebc, st_s, st_d, zb, acc, semg, sems, ebsem) = \
        args[nin + 1:nin + 10]
    rbufs = args[nin + 10:nin + 14]
    sixb = args[nin + 14:nin + 18]

    core = lax.axis_index("c")
    sub = lax.axis_index("s")
    zeros16 = jnp.zeros((16,), jnp.float32)

    def zi(k, _):
        zb[k, pl.ds(0, 16)] = zeros16
        return 0
    lax.fori_loop(0, _ZR, zi, 0)

    for p in range(2):
        slab = 2 * core + p
        for (ridx, cidx, bidir, nrowp, wb_base, d0, s0, d1, s1) in phases:
            rps = nrowp // _NSUB
            pad_slot = nrowp
            r_ref = idx_refs[ridx]
            c_ref = idx_refs[cidx]
            epb = 2 * _B if bidir else _B
            ngrp = epb // _G

            def za(q, _):
                pltpu.sync_copy(zb, acc.at[pl.ds(sub * rps + q * _ZR, _ZR)])
                return 0
            lax.fori_loop(0, rps // _ZR, za, 0)

            @pl.when(sub == 0)
            def _():
                pltpu.sync_copy(zb.at[pl.ds(0, 1)],
                                acc.at[pl.ds(pad_slot, 1)])
            plsc.subcore_barrier()

            eps_ = r_ref.shape[0] // _NSUB
            base = sub * eps_
            nblk = eps_ // _B

            pltpu.async_copy(r_ref.at[pl.ds(base, _B)], ebr.at[0], ebsem)
            pltpu.async_copy(c_ref.at[pl.ds(base, _B)], ebc.at[0], ebsem)

            def blk(b, _, r_ref=r_ref, c_ref=c_ref, base=base, bidir=bidir,
                    d0=d0, s0=s0, d1=d1, s1=s1, pad_slot=pad_slot,
                    ngrp=ngrp, slab=slab, nblk=nblk):
                par = lax.rem(b, 2)
                pltpu.make_async_copy(
                    r_ref.at[pl.ds(base + b * _B, _B)], ebr.at[par],
                    ebsem).wait()
                pltpu.make_async_copy(
                    c_ref.at[pl.ds(base + b * _B, _B)], ebc.at[par],
                    ebsem).wait()

                @pl.when(b + 1 < nblk)
                def _():
                    nb = base + (b + 1) * _B
                    pltpu.async_copy(r_ref.at[pl.ds(nb, _B)],
                                     ebr.at[1 - par], ebsem)
                    pltpu.async_copy(c_ref.at[pl.ds(nb, _B)],
                                     ebc.at[1 - par], ebsem)

                def vreg(k, _2):
                    r = ebr[par, pl.ds(k * 16, 16)]
                    c = ebc[par, pl.ds(k * 16, 16)]
                    ok = r < _SENT
                    pad = pad_slot + ((16 * k + lax.iota(jnp.int32, 16))
                                      & 2047)
                    if bidir:
                        st_d[pl.ds(32 * k, 16)] = jnp.where(ok, r + d0, pad)
                        st_s[pl.ds(32 * k, 16)] = jnp.where(ok, c + s0, 0)
                        st_d[pl.ds(32 * k + 16, 16)] = jnp.where(
                            ok, c + d1, pad)
                        st_s[pl.ds(32 * k + 16, 16)] = jnp.where(
                            ok, r + s1, 0)
                    else:
                        st_d[pl.ds(16 * k, 16)] = jnp.where(ok, r + d0, pad)
                        st_s[pl.ds(16 * k, 16)] = jnp.where(ok, c + s0, 0)
                    return 0
                lax.fori_loop(0, _B // 16, vreg, 0)

                gets = [None] * ngrp
                puts = [None] * ngrp
                for q in range(ngrp + 2):
                    if q < ngrp:
                        sl = q % 4
                        if q >= 4:
                            puts[q - 4].wait()
                        gets[q] = pltpu.async_copy(
                            table.at[slab].at[st_s.at[pl.ds(q * _G, _G)]],
                            rbufs[sl], semg)
                        for t in range(_G // 16):
                            sixb[sl][pl.ds(t * 16, 16)] = st_d[
                                pl.ds(q * _G + t * 16, 16)]
                    if q >= 2:
                        j = q - 2
                        sl2 = j % 4
                        gets[j].wait()
                        puts[j] = pltpu.async_copy(
                            rbufs[sl2], acc.at[sixb[sl2]], sems, add=True)
                for j in range(ngrp - 4, ngrp):
                    puts[j].wait()
                return 0
            lax.fori_loop(0, eps_ // _B, blk, 0)

            plsc.subcore_barrier()
            pltpu.sync_copy(
                acc.at[pl.ds(sub * rps, rps)],
                out.at[pl.ds(wb_base + sub * rps, rps),
                       pl.ds(_W * slab, _W)])
            plsc.subcore_barrier()


def _sc_spmm(table3, edge_arrays, phases, acc_rows, n_out):
    mesh = plsc.VectorSubcoreMesh(core_axis_name="c", subcore_axis_name="s")
    k = pl.kernel(
        functools.partial(_spmm_body, phases, acc_rows),
        out_type=jax.ShapeDtypeStruct((n_out, _D), jnp.float32),
        mesh=mesh,
        compiler_params=pltpu.CompilerParams(use_tc_tiling_on_sc=False),
        scratch_types=[
            pltpu.VMEM((2, _B), jnp.int32),
            pltpu.VMEM((2, _B), jnp.int32),
            pltpu.VMEM((2 * _B,), jnp.int32),
            pltpu.VMEM((2 * _B,), jnp.int32),
            pltpu.VMEM((_ZR, _W), jnp.float32),
            pltpu.VMEM_SHARED((acc_rows, _W), jnp.float32),
            pltpu.SemaphoreType.DMA,
            pltpu.SemaphoreType.DMA,
            pltpu.SemaphoreType.DMA,
        ] + [pltpu.VMEM((_G, _W), jnp.float32)] * 4
          + [pltpu.VMEM((_G,), jnp.int32)] * 4,
    )
    return k(table3, *edge_arrays)


def _deg_body(ui_u, ui_i, ub_u, ub_b, bi_b, ones8, out, st_d, ebr, onesb,
              zb8, acc, sems, ebsem, *sixb):
    core = lax.axis_index("c")
    sub = lax.axis_index("s")
    rps = _DEG_CHUNK // _NSUB

    pltpu.sync_copy(ones8.at[0].at[pl.ds(0, _GD)], onesb)
    pltpu.sync_copy(ones8.at[1].at[pl.ds(0, _ZR)], zb8)

    for chunk in range(3):
        lo = chunk * _DEG_CHUNK

        def za(q, _):
            pltpu.sync_copy(zb8, acc.at[pl.ds(sub * rps + q * _ZR, _ZR)])
            return 0
        lax.fori_loop(0, rps // _ZR, za, 0)

        @pl.when(sub == 0)
        def _():
            pltpu.sync_copy(zb8.at[pl.ds(0, 1)],
                            acc.at[pl.ds(_DEG_CHUNK, 1)])
        plsc.subcore_barrier()

        def count(idx_ref, seg_off):
            eps_ = idx_ref.shape[0] // _NSUB
            base = sub * eps_
            nblk = eps_ // _B
            ngrp = _B // _GD

            pltpu.async_copy(idx_ref.at[pl.ds(base, _B)], ebr.at[0], ebsem)

            def blk(b, _, idx_ref=idx_ref, seg_off=seg_off, base=base,
                    nblk=nblk, ngrp=ngrp):
                par = lax.rem(b, 2)
                pltpu.make_async_copy(
                    idx_ref.at[pl.ds(base + b * _B, _B)], ebr.at[par],
                    ebsem).wait()

                @pl.when(b + 1 < nblk)
                def _():
                    pltpu.async_copy(
                        idx_ref.at[pl.ds(base + (b + 1) * _B, _B)],
                        ebr.at[1 - par], ebsem)

                def vreg(k, _2):
                    v = ebr[par, pl.ds(k * 16, 16)]
                    d = v + (seg_off - lo)
                    ok = (v < _SENT) & (d >= 0) & (d < _DEG_CHUNK)
                    st_d[pl.ds(16 * k, 16)] = jnp.where(
                        ok, d, _DEG_CHUNK +
                        ((v + 16 * k + lax.iota(jnp.int32, 16)) & 2047))
                    return 0
                lax.fori_loop(0, _B // 16, vreg, 0)

                puts = [None] * ngrp
                for q in range(ngrp):
                    sl = q % 4
                    if q >= 4:
                        puts[q - 4].wait()
                    for t in range(_GD // 16):
                        sixb[sl][pl.ds(t * 16, 16)] = st_d[
                            pl.ds(q * _GD + t * 16, 16)]
                    puts[q] = pltpu.async_copy(
                        onesb, acc.at[sixb[sl]], sems, add=True)
                for j in range(ngrp - 4, ngrp):
                    puts[j].wait()
                return 0
            lax.fori_loop(0, nblk, blk, 0)

        @pl.when(core == 0)
        def _():
            count(ui_u, _DSEG0[0])
            count(ui_i, _DSEG0[1])

        @pl.when(core == 1)
        def _():
            count(ub_u, _DSEG1[0])
            count(ub_b, _DSEG1[1])
            count(bi_b, _DSEG1[2])

        plsc.subcore_barrier()
        pltpu.sync_copy(acc.at[pl.ds(sub * rps, rps)],
                        out.at[core].at[pl.ds(lo + sub * rps, rps)])
        plsc.subcore_barrier()


def _sc_degrees(ui_u, ui_i, ub_u, ub_b, bi_b, ones8):
    mesh = plsc.VectorSubcoreMesh(core_axis_name="c", subcore_axis_name="s")
    k = pl.kernel(
        _deg_body,
        out_type=jax.ShapeDtypeStruct((2, _DEG_ROWS, _DW), jnp.float32),
        mesh=mesh,
        compiler_params=pltpu.CompilerParams(use_tc_tiling_on_sc=False),
        scratch_types=[
            pltpu.VMEM((_B,), jnp.int32),
            pltpu.VMEM((2, _B), jnp.int32),
            pltpu.VMEM((_GD, _DW), jnp.float32),
            pltpu.VMEM((_ZR, _DW), jnp.float32),
            pltpu.VMEM_SHARED((_DEG_CHUNK + 2048, _DW), jnp.float32),
            pltpu.SemaphoreType.DMA,
            pltpu.SemaphoreType.DMA,
        ] + [pltpu.VMEM((_GD,), jnp.int32)] * 4,
    )
    return k(ui_u, ui_i, ub_u, ub_b, bi_b, ones8)


def _to_slabs(x):
    return x.reshape(x.shape[0], _NSLAB, _W).transpose(1, 0, 2)


def _from_slabs(x3):
    return x3.transpose(1, 0, 2).reshape(x3.shape[1], _D)


_PHASES_MAIN = [
    (0, 1, True, 90112, 0, 0, _OFF_I, _OFF_I, 0),
    (2, 3, True, 73728, _OFF_U2, 0, _OFF_B, _NU, _OFF_U2),
]

_PHASES_BI = [
    (0, 1, False, 20480, 0, 0, _OFF_I, None, None),
]


def kernel(users_feature, bundles_feature, items_feature, ui_u, ui_i, ub_u, ub_b, bi_b, bi_i):
    ui_u, ui_i = _pad_edges(ui_u), _pad_edges(ui_i)
    ub_u, ub_b = _pad_edges(ub_u), _pad_edges(ub_b)
    bi_b, bi_i = _pad_edges(bi_b), _pad_edges(bi_i)

    z8 = users_feature[:8192, :_DW] * 0.0
    ones8 = jnp.stack([z8 + 1.0, z8])
    degw = _sc_degrees(ui_u, ui_i, ub_u, ub_b, bi_b, ones8)
    d0 = degw[0, :, 0]
    d1 = degw[1, :, 0]
    s_all = _ew_1d(_scale_body, jnp.concatenate([
        d0[0:_NU], d0[_DSEG0[1]:_DSEG0[1] + _NI],
        d1[0:_NU], d1[_DSEG1[1]:_DSEG1[1] + _NB],
        jnp.zeros((_NPAD - _NTOT,), jnp.float32),
    ]))
    inv_bs = _ew_1d(_inv_body, jnp.concatenate([
        d1[_DSEG1[2]:_DSEG1[2] + _NB],
        jnp.zeros((20480 - _NB,), jnp.float32),
    ]), rows=160)

    table0 = jnp.concatenate([
        users_feature, items_feature, users_feature, bundles_feature,
        jnp.zeros((_NPAD - _NTOT, _D), jnp.float32),
    ])
    acc = table0
    g = _rowscale(table0, s_all)

    for i in range(2):
        h = _sc_spmm(_to_slabs(g), [ui_u, ui_i, ub_u, ub_b], _PHASES_MAIN,
                     acc_rows=92160, n_out=_NPAD)
        acc, g = _layer_update(h, s_all, acc, 1.0 / (i + 2))

    hb = _sc_spmm(_to_slabs(acc), [bi_b, bi_i], _PHASES_BI,
                  acc_rows=22528, n_out=20480)
    il_bundles = _rowscale(hb, inv_bs)[:_NB]

    return (acc[:_NU], acc[_OFF_U2:_OFF_U2 + _NU], il_bundles,
            acc[_OFF_B:_OFF_B + _NB])
